# baseline (speedup 1.0000x reference)
import functools

import jax
import jax.numpy as jnp
from jax import lax
from jax.experimental import pallas as pl
from jax.experimental.pallas import tpu as pltpu
from jax.experimental.pallas import tpu_sc as plsc

BATCH = 4096
SEQ = 200
EMBED = 64
NUM_WORKERS = 32
BPW = BATCH // NUM_WORKERS
NSLOT = 4
NROUND = BPW // NSLOT

_mesh = plsc.VectorSubcoreMesh(core_axis_name="c", subcore_axis_name="s")


@functools.partial(
    pl.kernel,
    out_type=jax.ShapeDtypeStruct((BATCH, SEQ, EMBED), jnp.float32),
    mesh=_mesh,
    compiler_params=pltpu.CompilerParams(use_tc_tiling_on_sc=False),
    scratch_types=[
        pltpu.VMEM((BPW, SEQ), jnp.int32),
        pltpu.VMEM((NSLOT, SEQ, EMBED), jnp.float32),
        pltpu.SemaphoreType.DMA((NSLOT,)),
        pltpu.SemaphoreType.DMA((NSLOT,)),
    ],
)
def _embed_sc(idx_hbm, table_hbm, out_hbm, idx_v, bufs, gsems, ssems):
    wid = lax.axis_index("s") * 2 + lax.axis_index("c")
    base = wid * BPW
    pltpu.sync_copy(idx_hbm.at[pl.ds(base, BPW)], idx_v)

    def body(r, carry):
        for s in range(NSLOT):
            b = r * NSLOT + s

            @pl.when(r > 0)
            def _():
                pltpu.make_async_copy(
                    bufs.at[s], out_hbm.at[base], ssems.at[s]
                ).wait()

            pltpu.async_copy(
                table_hbm.at[idx_v.at[b]], bufs.at[s], gsems.at[s])
        for s in range(NSLOT):
            b = r * NSLOT + s
            pltpu.make_async_copy(
                table_hbm.at[pl.ds(0, SEQ)], bufs.at[s], gsems.at[s]
            ).wait()
            pltpu.async_copy(bufs.at[s], out_hbm.at[base + b], ssems.at[s])
        return carry

    lax.fori_loop(0, NROUND, body, 0)
    for s in range(NSLOT):
        pltpu.make_async_copy(bufs.at[s], out_hbm.at[base], ssems.at[s]).wait()


def kernel(word_input, weight_all):
    return _embed_sc(word_input.astype(jnp.int32), weight_all)


# speedup vs baseline: 1.0027x; 1.0027x over previous
import functools

import jax
import jax.numpy as jnp
from jax import lax
from jax.experimental import pallas as pl
from jax.experimental.pallas import tpu as pltpu
from jax.experimental.pallas import tpu_sc as plsc

BATCH = 4096
SEQ = 200
EMBED = 64
N = BATCH * SEQ
NUM_WORKERS = 32
PER_W = N // NUM_WORKERS
NSLOT = 8
CHUNK = 128
NROUND = PER_W // (NSLOT * CHUNK)

_mesh = plsc.VectorSubcoreMesh(core_axis_name="c", subcore_axis_name="s")


@functools.partial(
    pl.kernel,
    out_type=jax.ShapeDtypeStruct((N, EMBED), jnp.float32),
    mesh=_mesh,
    compiler_params=pltpu.CompilerParams(use_tc_tiling_on_sc=False),
    scratch_types=[
        pltpu.VMEM((PER_W,), jnp.int32),
        pltpu.VMEM((NSLOT, CHUNK, EMBED), jnp.float32),
        pltpu.SemaphoreType.DMA((NSLOT,)),
        pltpu.SemaphoreType.DMA((NSLOT,)),
    ],
)
def _embed_sc(idx_hbm, table_hbm, out_hbm, idx_v, bufs, gsems, ssems):
    wid = lax.axis_index("s") * 2 + lax.axis_index("c")
    base = wid * PER_W
    pltpu.sync_copy(idx_hbm.at[pl.ds(base, PER_W)], idx_v)

    def body(r, carry):
        for s in range(NSLOT):
            chunk_off = pl.multiple_of((r * NSLOT + s) * CHUNK, 8)

            @pl.when(r > 0)
            def _():
                pltpu.make_async_copy(
                    bufs.at[s], out_hbm.at[pl.ds(base, CHUNK)], ssems.at[s]
                ).wait()

            pltpu.async_copy(
                table_hbm.at[idx_v.at[pl.ds(chunk_off, CHUNK)]],
                bufs.at[s], gsems.at[s])
        for s in range(NSLOT):
            chunk_off = pl.multiple_of((r * NSLOT + s) * CHUNK, 8)
            pltpu.make_async_copy(
                table_hbm.at[pl.ds(0, CHUNK)], bufs.at[s], gsems.at[s]
            ).wait()
            pltpu.async_copy(
                bufs.at[s], out_hbm.at[pl.ds(base + chunk_off, CHUNK)],
                ssems.at[s])
        return carry

    lax.fori_loop(0, NROUND, body, 0)
    for s in range(NSLOT):
        pltpu.make_async_copy(
            bufs.at[s], out_hbm.at[pl.ds(base, CHUNK)], ssems.at[s]
        ).wait()


def kernel(word_input, weight_all):
    idx = word_input.reshape(N).astype(jnp.int32)
    out = _embed_sc(idx, weight_all)
    return out.reshape(BATCH, SEQ, EMBED)
